# baseline (speedup 1.0000x reference)
import functools

import jax
import jax.numpy as jnp
from jax import lax
from jax.experimental import pallas as pl
from jax.experimental.pallas import tpu as pltpu
from jax.experimental.pallas import tpu_sc as plsc

_N = 10000
_E = 160000
_DF = 128
_DE = 16
_H = 32
_NG = 256
_DIMS = [(32, 32), (32, 32), (32, 16)]

_NC, _NS = 2, 16
_NW = _NC * _NS
_CH = 128
_NCHUNKS = _E // _CH
_BASE_CHUNKS = _NCHUNKS // _NW
_EXTRA = _NCHUNKS - _BASE_CHUNKS * _NW
_NPS = _N // _NS


def _gather_body(tbl_hbm, idx_hbm, out_hbm, idx_v, row_v, sem):
    wid = lax.axis_index("s") * _NC + lax.axis_index("c")
    nchunks = jnp.where(wid < _EXTRA, _BASE_CHUNKS + 1, _BASE_CHUNKS)

    @pl.loop(0, nchunks)
    def _(t):
        chunk = wid + t * _NW
        off = pl.multiple_of(chunk * _CH, 1024)
        pltpu.sync_copy(idx_hbm.at[pl.ds(off, _CH)], idx_v)
        pltpu.async_copy(tbl_hbm.at[idx_v], row_v, sem).wait()
        pltpu.sync_copy(row_v, out_hbm.at[pl.ds(off, _CH)])


@functools.lru_cache(maxsize=None)
def _sc_mesh():
    return plsc.VectorSubcoreMesh(
        core_axis_name="c", subcore_axis_name="s",
        num_cores=_NC, num_subcores=_NS)


@functools.lru_cache(maxsize=None)
def _gather_kernel():
    return pl.kernel(
        _gather_body,
        out_type=jax.ShapeDtypeStruct((_E, _H), jnp.float32),
        mesh=_sc_mesh(),
        compiler_params=pltpu.CompilerParams(use_tc_tiling_on_sc=False),
        scratch_types=[
            pltpu.VMEM((_CH,), jnp.int32),
            pltpu.VMEM((_CH, _H), jnp.float32),
            pltpu.SemaphoreType.DMA,
        ],
    )


def _gather(tbl, idx):
    return _gather_kernel()(tbl, idx)


def _make_scatter(width):
    def body(msg_hbm, idx_hbm, zero_hbm, out_hbm, idx_v, row_v, acc_sh):
        cid = lax.axis_index("c")
        sid = lax.axis_index("s")
        wid = sid * _NC + cid
        nchunks = jnp.where(wid < _EXTRA, _BASE_CHUNKS + 1, _BASE_CHUNKS)
        slab = pl.ds(sid * _NPS, _NPS)
        pltpu.sync_copy(zero_hbm.at[slab], acc_sh.at[slab])
        plsc.subcore_barrier()

        @pl.loop(0, nchunks)
        def _(t):
            chunk = wid + t * _NW
            off = pl.multiple_of(chunk * _CH, 1024)
            pltpu.sync_copy(idx_hbm.at[pl.ds(off, _CH)], idx_v)
            pltpu.sync_copy(msg_hbm.at[pl.ds(off, _CH)], row_v)
            pltpu.sync_copy(row_v, acc_sh.at[idx_v], add=True)

        plsc.subcore_barrier()
        pltpu.sync_copy(acc_sh.at[slab], out_hbm.at[cid].at[slab])

    return pl.kernel(
        body,
        out_type=jax.ShapeDtypeStruct((_NC, _N, width), jnp.float32),
        mesh=_sc_mesh(),
        compiler_params=pltpu.CompilerParams(use_tc_tiling_on_sc=False),
        scratch_types=[
            pltpu.VMEM((_CH,), jnp.int32),
            pltpu.VMEM((_CH, width), jnp.float32),
            pltpu.VMEM_SHARED((_N, width), jnp.float32),
        ],
    )


_make_scatter = functools.lru_cache(maxsize=None)(_make_scatter)


def _scatter32(msg, dst, zeros):
    return _make_scatter(32)(msg, dst, zeros)


def _scatter16(msg, dst, zeros):
    return _make_scatter(16)(msg, dst, zeros)


def _cnt_body(idx_hbm, ones_hbm, zero_hbm, out_hbm, idx_v, row_v, acc_sh):
    cid = lax.axis_index("c")
    sid = lax.axis_index("s")
    wid = sid * _NC + cid
    nchunks = jnp.where(wid < _EXTRA, _BASE_CHUNKS + 1, _BASE_CHUNKS)
    slab = pl.ds(sid * _NPS, _NPS)
    pltpu.sync_copy(zero_hbm.at[slab], acc_sh.at[slab])
    pltpu.sync_copy(ones_hbm, row_v)
    plsc.subcore_barrier()

    @pl.loop(0, nchunks)
    def _(t):
        chunk = wid + t * _NW
        off = pl.multiple_of(chunk * _CH, 1024)
        pltpu.sync_copy(idx_hbm.at[pl.ds(off, _CH)], idx_v)
        pltpu.sync_copy(row_v, acc_sh.at[idx_v], add=True)

    plsc.subcore_barrier()
    pltpu.sync_copy(acc_sh.at[slab], out_hbm.at[cid].at[slab])


@functools.lru_cache(maxsize=None)
def _cnt_kernel():
    return pl.kernel(
        _cnt_body,
        out_type=jax.ShapeDtypeStruct((_NC, _N, 16), jnp.float32),
        mesh=_sc_mesh(),
        compiler_params=pltpu.CompilerParams(use_tc_tiling_on_sc=False),
        scratch_types=[
            pltpu.VMEM((_CH,), jnp.int32),
            pltpu.VMEM((_CH, 16), jnp.float32),
            pltpu.VMEM_SHARED((_N, 16), jnp.float32),
        ],
    )


def _cnt(idx, ones_row, zeros):
    return _cnt_kernel()(idx, ones_row, zeros)


def _proj_body(x_ref, w_ref, b_ref, o_ref):
    o_ref[...] = (
        jnp.dot(x_ref[...], w_ref[...], preferred_element_type=jnp.float32)
        + b_ref[...]
    )


def _node_proj(x, w, b):
    return pl.pallas_call(
        _proj_body,
        out_shape=jax.ShapeDtypeStruct((_N, _H), jnp.float32),
        grid=(10,),
        in_specs=[
            pl.BlockSpec((_N // 10, _DF), lambda i: (i, 0)),
            pl.BlockSpec((_DF, _H), lambda i: (0, 0)),
            pl.BlockSpec((1, _H), lambda i: (0, 0)),
        ],
        out_specs=pl.BlockSpec((_N // 10, _H), lambda i: (i, 0)),
    )(x, w, b)


_MB = 256


def _make_msg_body(din, dout):
    lanes_per_grp = 128 // dout
    ngrp = din // lanes_per_grp

    def body(ea_ref, xj_ref, w1_ref, b1_ref, w2_ref, b2_ref, o_ref):
        hid = jnp.maximum(
            jnp.dot(ea_ref[...], w1_ref[...], preferred_element_type=jnp.float32)
            + b1_ref[...],
            0.0,
        )
        xj = xj_ref[...]
        acc = jnp.zeros((_MB, dout), jnp.float32)
        for g in range(ngrp):
            weg = (
                jnp.dot(
                    hid,
                    w2_ref[:, g * 128:(g + 1) * 128],
                    preferred_element_type=jnp.float32,
                )
                + b2_ref[:, g * 128:(g + 1) * 128]
            )
            for j in range(lanes_per_grp):
                i = g * lanes_per_grp + j
                acc = acc + xj[:, i:i + 1] * weg[:, j * dout:(j + 1) * dout]
        o_ref[...] = acc

    return body


def _msg(ea, xj, w1f, b1f, w2, b2, din, dout):
    return pl.pallas_call(
        _make_msg_body(din, dout),
        out_shape=jax.ShapeDtypeStruct((_E, dout), jnp.float32),
        grid=(_E // _MB,),
        in_specs=[
            pl.BlockSpec((_MB, _DE), lambda e: (e, 0)),
            pl.BlockSpec((_MB, din), lambda e: (e, 0)),
            pl.BlockSpec((_DE, 2 * _H), lambda e: (0, 0)),
            pl.BlockSpec((1, 2 * _H), lambda e: (0, 0)),
            pl.BlockSpec((2 * _H, din * dout), lambda e: (0, 0)),
            pl.BlockSpec((1, din * dout), lambda e: (0, 0)),
        ],
        out_specs=pl.BlockSpec((_MB, dout), lambda e: (e, 0)),
    )(ea, xj, w1f, b1f, w2, b2)


def _make_upd_body(dout):
    def body(s2_ref, c2_ref, h_ref, root_ref, bias_ref, gam_ref, bet_ref, o_ref):
        s = s2_ref[0] + s2_ref[1]
        cnt = c2_ref[0][:, 0:1] + c2_ref[1][:, 0:1]
        hn = (
            s / jnp.maximum(cnt, 1.0)
            + jnp.dot(h_ref[...], root_ref[...], preferred_element_type=jnp.float32)
            + bias_ref[...]
        )
        mu = jnp.mean(hn, axis=0, keepdims=True)
        var = jnp.mean((hn - mu) * (hn - mu), axis=0, keepdims=True)
        bn = (hn - mu) * lax.rsqrt(var + 1e-5) * gam_ref[...] + bet_ref[...]
        o_ref[...] = jnp.maximum(bn, 0.0)

    return body


def _update(s2, c2, h, root, bias, gamma, beta, din, dout):
    return pl.pallas_call(
        _make_upd_body(dout),
        out_shape=jax.ShapeDtypeStruct((_N, dout), jnp.float32),
        in_specs=[
            pl.BlockSpec((_NC, _N, dout), lambda: (0, 0, 0)),
            pl.BlockSpec((_NC, _N, 16), lambda: (0, 0, 0)),
            pl.BlockSpec((_N, din), lambda: (0, 0)),
            pl.BlockSpec((din, dout), lambda: (0, 0)),
            pl.BlockSpec((1, dout), lambda: (0, 0)),
            pl.BlockSpec((1, dout), lambda: (0, 0)),
            pl.BlockSpec((1, dout), lambda: (0, 0)),
        ],
        out_specs=pl.BlockSpec((_N, dout), lambda: (0, 0)),
    )(s2, c2, h, root, bias, gamma, beta)


def _pool_body(h_ref, b_ref, fcw_ref, fcb_ref, o_ref):
    h = h_ref[...]
    onehot = (b_ref[...] == lax.broadcasted_iota(jnp.int32, (1, _NG), 1)).astype(
        jnp.float32
    )
    dn = (((0,), (0,)), ((), ()))
    ps = lax.dot_general(onehot, h, dn, preferred_element_type=jnp.float32)
    pc = lax.dot_general(
        onehot, jnp.ones((_N, 1), jnp.float32), dn, preferred_element_type=jnp.float32
    )
    pooled = ps / jnp.maximum(pc, 1.0)
    logits = (
        jnp.dot(pooled, fcw_ref[...], preferred_element_type=jnp.float32)
        + fcb_ref[...]
    )
    o_ref[...] = 1.0 / (1.0 + jnp.exp(-logits))


def _pool(h, batch2d, fcw, fcb):
    return pl.pallas_call(
        _pool_body,
        out_shape=jax.ShapeDtypeStruct((_NG, 12), jnp.float32),
        in_specs=[
            pl.BlockSpec((_N, 16), lambda: (0, 0)),
            pl.BlockSpec((_N, 1), lambda: (0, 0)),
            pl.BlockSpec((16, 12), lambda: (0, 0)),
            pl.BlockSpec((1, 12), lambda: (0, 0)),
        ],
        out_specs=pl.BlockSpec((_NG, 12), lambda: (0, 0)),
    )(h, batch2d, fcw, fcb)


def kernel(x, edge_index, edge_attr, batch, params):
    src = edge_index[0]
    dst = edge_index[1]

    h = _node_proj(x, params['node_W'], params['node_b'].reshape(1, _H))

    zeros16 = jnp.zeros((_N, 16), jnp.float32)
    ones_row = jnp.ones((_CH, 16), jnp.float32)
    c2 = _cnt(dst, ones_row, zeros16)

    zeros32 = jnp.zeros((_N, 32), jnp.float32)
    for i, (din, dout) in enumerate(_DIMS):
        lp = params['layers'][i]
        w1f = params['edge_W'] @ lp['W1']
        b1f = (params['edge_b'] @ lp['W1'] + lp['b1']).reshape(1, 2 * _H)
        xj = _gather(h, src)
        msg = _msg(edge_attr, xj, w1f, b1f, lp['W2'],
                   lp['b2'].reshape(1, din * dout), din, dout)
        scat = _scatter32 if dout == 32 else _scatter16
        s2 = scat(msg, dst, zeros32 if dout == 32 else zeros16)
        h = _update(s2, c2, h, lp['root'], lp['bias'].reshape(1, dout),
                    lp['gamma'].reshape(1, dout), lp['beta'].reshape(1, dout),
                    din, dout)

    return _pool(h, batch.reshape(_N, 1), params['fc_W'],
                 params['fc_b'].reshape(1, 12))


# speedup vs baseline: 2.6486x; 2.6486x over previous
import functools

import jax
import jax.numpy as jnp
import numpy as np
from jax import lax
from jax.experimental import pallas as pl
from jax.experimental.pallas import tpu as pltpu
from jax.experimental.pallas import tpu_sc as plsc

_N = 10000
_E = 160000
_DF = 128
_DE = 16
_H = 32
_NG = 256
_DIMS = [(32, 32), (32, 32), (32, 16)]

_NC, _NS = 2, 16
_NW = _NC * _NS
_CH = 128
_NCHUNKS = _E // _CH
_BASE_CHUNKS = _NCHUNKS // _NW
_EXTRA = _NCHUNKS - _BASE_CHUNKS * _NW
_NPS = _N // _NS


def _gather_body(tbl_hbm, idx_hbm, out_hbm, idx_v, row_v, sem):
    wid = lax.axis_index("s") * _NC + lax.axis_index("c")
    nchunks = jnp.where(wid < _EXTRA, _BASE_CHUNKS + 1, _BASE_CHUNKS)

    @pl.loop(0, nchunks)
    def _(t):
        chunk = wid + t * _NW
        off = pl.multiple_of(chunk * _CH, 1024)
        pltpu.sync_copy(idx_hbm.at[pl.ds(off, _CH)], idx_v)
        pltpu.async_copy(tbl_hbm.at[idx_v], row_v, sem).wait()
        pltpu.sync_copy(row_v, out_hbm.at[pl.ds(off, _CH)])


@functools.lru_cache(maxsize=None)
def _sc_mesh():
    return plsc.VectorSubcoreMesh(
        core_axis_name="c", subcore_axis_name="s",
        num_cores=_NC, num_subcores=_NS)


@functools.lru_cache(maxsize=None)
def _gather_kernel():
    return pl.kernel(
        _gather_body,
        out_type=jax.ShapeDtypeStruct((_E, _H), jnp.float32),
        mesh=_sc_mesh(),
        compiler_params=pltpu.CompilerParams(use_tc_tiling_on_sc=False),
        scratch_types=[
            pltpu.VMEM((_CH,), jnp.int32),
            pltpu.VMEM((_CH, _H), jnp.float32),
            pltpu.SemaphoreType.DMA,
        ],
    )


def _gather(tbl, idx):
    return _gather_kernel()(tbl, idx)


def _make_scatter(width):
    def body(msg_hbm, idx_hbm, zero_hbm, out_hbm, idx_v, row_v, acc_sh):
        cid = lax.axis_index("c")
        sid = lax.axis_index("s")
        wid = sid * _NC + cid
        nchunks = jnp.where(wid < _EXTRA, _BASE_CHUNKS + 1, _BASE_CHUNKS)
        slab = pl.ds(sid * _NPS, _NPS)
        pltpu.sync_copy(zero_hbm.at[slab], acc_sh.at[slab])
        plsc.subcore_barrier()

        @pl.loop(0, nchunks)
        def _(t):
            chunk = wid + t * _NW
            off = pl.multiple_of(chunk * _CH, 1024)
            pltpu.sync_copy(idx_hbm.at[pl.ds(off, _CH)], idx_v)
            pltpu.sync_copy(msg_hbm.at[pl.ds(off, _CH)], row_v)
            pltpu.sync_copy(row_v, acc_sh.at[idx_v], add=True)

        plsc.subcore_barrier()
        pltpu.sync_copy(acc_sh.at[slab], out_hbm.at[cid].at[slab])

    return pl.kernel(
        body,
        out_type=jax.ShapeDtypeStruct((_NC, _N, width), jnp.float32),
        mesh=_sc_mesh(),
        compiler_params=pltpu.CompilerParams(use_tc_tiling_on_sc=False),
        scratch_types=[
            pltpu.VMEM((_CH,), jnp.int32),
            pltpu.VMEM((_CH, width), jnp.float32),
            pltpu.VMEM_SHARED((_N, width), jnp.float32),
        ],
    )


_make_scatter = functools.lru_cache(maxsize=None)(_make_scatter)


def _scatter32(msg, dst, zeros):
    return _make_scatter(32)(msg, dst, zeros)


def _scatter16(msg, dst, zeros):
    return _make_scatter(16)(msg, dst, zeros)


def _cnt_body(idx_hbm, ones_hbm, zero_hbm, out_hbm, idx_v, row_v, acc_sh):
    cid = lax.axis_index("c")
    sid = lax.axis_index("s")
    wid = sid * _NC + cid
    nchunks = jnp.where(wid < _EXTRA, _BASE_CHUNKS + 1, _BASE_CHUNKS)
    slab = pl.ds(sid * _NPS, _NPS)
    pltpu.sync_copy(zero_hbm.at[slab], acc_sh.at[slab])
    pltpu.sync_copy(ones_hbm, row_v)
    plsc.subcore_barrier()

    @pl.loop(0, nchunks)
    def _(t):
        chunk = wid + t * _NW
        off = pl.multiple_of(chunk * _CH, 1024)
        pltpu.sync_copy(idx_hbm.at[pl.ds(off, _CH)], idx_v)
        pltpu.sync_copy(row_v, acc_sh.at[idx_v], add=True)

    plsc.subcore_barrier()
    pltpu.sync_copy(acc_sh.at[slab], out_hbm.at[cid].at[slab])


@functools.lru_cache(maxsize=None)
def _cnt_kernel():
    return pl.kernel(
        _cnt_body,
        out_type=jax.ShapeDtypeStruct((_NC, _N, 16), jnp.float32),
        mesh=_sc_mesh(),
        compiler_params=pltpu.CompilerParams(use_tc_tiling_on_sc=False),
        scratch_types=[
            pltpu.VMEM((_CH,), jnp.int32),
            pltpu.VMEM((_CH, 16), jnp.float32),
            pltpu.VMEM_SHARED((_N, 16), jnp.float32),
        ],
    )


def _cnt(idx, ones_row, zeros):
    return _cnt_kernel()(idx, ones_row, zeros)


def _proj_body(x_ref, w_ref, b_ref, o_ref):
    o_ref[...] = (
        jnp.dot(x_ref[...], w_ref[...], preferred_element_type=jnp.float32)
        + b_ref[...]
    )


def _node_proj(x, w, b):
    return pl.pallas_call(
        _proj_body,
        out_shape=jax.ShapeDtypeStruct((_N, _H), jnp.float32),
        grid=(10,),
        in_specs=[
            pl.BlockSpec((_N // 10, _DF), lambda i: (i, 0)),
            pl.BlockSpec((_DF, _H), lambda i: (0, 0)),
            pl.BlockSpec((1, _H), lambda i: (0, 0)),
        ],
        out_specs=pl.BlockSpec((_N // 10, _H), lambda i: (i, 0)),
    )(x, w, b)


_MB = 256


def _make_msg_body(din, dout):
    def body(ea_ref, xj_ref, w1_ref, b1_ref, w2_ref, b2_ref, r_ref, o_ref):
        hid = jnp.maximum(
            jnp.dot(ea_ref[...], w1_ref[...], preferred_element_type=jnp.float32)
            + b1_ref[...],
            0.0,
        )
        we = (
            jnp.dot(hid, w2_ref[...], preferred_element_type=jnp.float32)
            + b2_ref[...]
        )
        x4 = jnp.dot(xj_ref[...], r_ref[...], preferred_element_type=jnp.float32)
        p = x4 * we
        w = din * dout
        while w > dout:
            w //= 2
            p = p[:, :w] + p[:, w:]
        o_ref[...] = p

    return body


def _msg(ea, xj, w1f, b1f, w2, b2, rmat, din, dout):
    return pl.pallas_call(
        _make_msg_body(din, dout),
        out_shape=jax.ShapeDtypeStruct((_E, dout), jnp.float32),
        grid=(_E // _MB,),
        in_specs=[
            pl.BlockSpec((_MB, _DE), lambda e: (e, 0)),
            pl.BlockSpec((_MB, din), lambda e: (e, 0)),
            pl.BlockSpec((_DE, 2 * _H), lambda e: (0, 0)),
            pl.BlockSpec((1, 2 * _H), lambda e: (0, 0)),
            pl.BlockSpec((2 * _H, din * dout), lambda e: (0, 0)),
            pl.BlockSpec((1, din * dout), lambda e: (0, 0)),
            pl.BlockSpec((din, din * dout), lambda e: (0, 0)),
        ],
        out_specs=pl.BlockSpec((_MB, dout), lambda e: (e, 0)),
    )(ea, xj, w1f, b1f, w2, b2, rmat)


def _make_upd_body(dout):
    def body(s2_ref, c2_ref, h_ref, root_ref, bias_ref, gam_ref, bet_ref, o_ref):
        s = s2_ref[0] + s2_ref[1]
        cnt = c2_ref[0][:, 0:1] + c2_ref[1][:, 0:1]
        hn = (
            s / jnp.maximum(cnt, 1.0)
            + jnp.dot(h_ref[...], root_ref[...], preferred_element_type=jnp.float32)
            + bias_ref[...]
        )
        mu = jnp.mean(hn, axis=0, keepdims=True)
        var = jnp.mean((hn - mu) * (hn - mu), axis=0, keepdims=True)
        bn = (hn - mu) * lax.rsqrt(var + 1e-5) * gam_ref[...] + bet_ref[...]
        o_ref[...] = jnp.maximum(bn, 0.0)

    return body


def _update(s2, c2, h, root, bias, gamma, beta, din, dout):
    return pl.pallas_call(
        _make_upd_body(dout),
        out_shape=jax.ShapeDtypeStruct((_N, dout), jnp.float32),
        in_specs=[
            pl.BlockSpec((_NC, _N, dout), lambda: (0, 0, 0)),
            pl.BlockSpec((_NC, _N, 16), lambda: (0, 0, 0)),
            pl.BlockSpec((_N, din), lambda: (0, 0)),
            pl.BlockSpec((din, dout), lambda: (0, 0)),
            pl.BlockSpec((1, dout), lambda: (0, 0)),
            pl.BlockSpec((1, dout), lambda: (0, 0)),
            pl.BlockSpec((1, dout), lambda: (0, 0)),
        ],
        out_specs=pl.BlockSpec((_N, dout), lambda: (0, 0)),
    )(s2, c2, h, root, bias, gamma, beta)


def _pool_body(h_ref, b_ref, fcw_ref, fcb_ref, o_ref):
    h = h_ref[...]
    onehot = (b_ref[...] == lax.broadcasted_iota(jnp.int32, (1, _NG), 1)).astype(
        jnp.float32
    )
    dn = (((0,), (0,)), ((), ()))
    ps = lax.dot_general(onehot, h, dn, preferred_element_type=jnp.float32)
    pc = lax.dot_general(
        onehot, jnp.ones((_N, 1), jnp.float32), dn, preferred_element_type=jnp.float32
    )
    pooled = ps / jnp.maximum(pc, 1.0)
    logits = (
        jnp.dot(pooled, fcw_ref[...], preferred_element_type=jnp.float32)
        + fcb_ref[...]
    )
    o_ref[...] = 1.0 / (1.0 + jnp.exp(-logits))


def _pool(h, batch2d, fcw, fcb):
    return pl.pallas_call(
        _pool_body,
        out_shape=jax.ShapeDtypeStruct((_NG, 12), jnp.float32),
        in_specs=[
            pl.BlockSpec((_N, 16), lambda: (0, 0)),
            pl.BlockSpec((_N, 1), lambda: (0, 0)),
            pl.BlockSpec((16, 12), lambda: (0, 0)),
            pl.BlockSpec((1, 12), lambda: (0, 0)),
        ],
        out_specs=pl.BlockSpec((_NG, 12), lambda: (0, 0)),
    )(h, batch2d, fcw, fcb)


def kernel(x, edge_index, edge_attr, batch, params):
    src = edge_index[0]
    dst = edge_index[1]

    h = _node_proj(x, params['node_W'], params['node_b'].reshape(1, _H))

    zeros16 = jnp.zeros((_N, 16), jnp.float32)
    ones_row = jnp.ones((_CH, 16), jnp.float32)
    c2 = _cnt(dst, ones_row, zeros16)

    zeros32 = jnp.zeros((_N, 32), jnp.float32)
    for i, (din, dout) in enumerate(_DIMS):
        lp = params['layers'][i]
        w1f = params['edge_W'] @ lp['W1']
        b1f = (params['edge_b'] @ lp['W1'] + lp['b1']).reshape(1, 2 * _H)
        xj = _gather(h, src)
        rmat = jnp.asarray(
            np.kron(np.eye(din, dtype=np.float32),
                    np.ones((1, dout), np.float32)))
        msg = _msg(edge_attr, xj, w1f, b1f, lp['W2'],
                   lp['b2'].reshape(1, din * dout), rmat, din, dout)
        scat = _scatter32 if dout == 32 else _scatter16
        s2 = scat(msg, dst, zeros32 if dout == 32 else zeros16)
        h = _update(s2, c2, h, lp['root'], lp['bias'].reshape(1, dout),
                    lp['gamma'].reshape(1, dout), lp['beta'].reshape(1, dout),
                    din, dout)

    return _pool(h, batch.reshape(_N, 1), params['fc_W'],
                 params['fc_b'].reshape(1, 12))


# speedup vs baseline: 4.2760x; 1.6145x over previous
import functools

import jax
import jax.numpy as jnp
import numpy as np
from jax import lax
from jax.experimental import pallas as pl
from jax.experimental.pallas import tpu as pltpu
from jax.experimental.pallas import tpu_sc as plsc

_N = 10000
_E = 160000
_DF = 128
_DE = 16
_H = 32
_NG = 256
_DIMS = [(32, 32), (32, 32), (32, 16)]

_NC, _NS = 2, 16
_NW = _NC * _NS
_CH = 128
_NCHUNKS = _E // _CH
_CPW = _NCHUNKS // _NW
_GK = 8
_NPS = _N // _NS


def _sc_group_loop(wid, fn):
    base = wid * _CPW
    for g in range(_CPW // _GK):
        fn(base + g * _GK, _GK)
    rem = _CPW % _GK
    fn(base + _CPW - rem, rem)

    @pl.when(wid < _NCHUNKS - _CPW * _NW)
    def _():
        fn(_CPW * _NW + wid, 1)


def _gather_body(tbl_hbm, idx_hbm, out_hbm, idx_v, row_v, sem):
    wid = lax.axis_index("s") * _NC + lax.axis_index("c")

    def fn(row0, k):
        pltpu.sync_copy(idx_hbm.at[pl.ds(row0, k)], idx_v.at[pl.ds(0, k)])
        descs = [
            pltpu.async_copy(
                tbl_hbm.at[idx_v.at[b]],
                row_v.at[pl.ds(b * _CH, _CH)],
                sem,
            )
            for b in range(k)
        ]
        for d in descs:
            d.wait()
        pltpu.sync_copy(
            row_v.at[pl.ds(0, k * _CH)],
            out_hbm.at[pl.ds(pl.multiple_of(row0 * _CH, _CH), k * _CH)],
        )

    _sc_group_loop(wid, fn)


@functools.lru_cache(maxsize=None)
def _sc_mesh():
    return plsc.VectorSubcoreMesh(
        core_axis_name="c", subcore_axis_name="s",
        num_cores=_NC, num_subcores=_NS)


@functools.lru_cache(maxsize=None)
def _gather_kernel():
    return pl.kernel(
        _gather_body,
        out_type=jax.ShapeDtypeStruct((_E, _H), jnp.float32),
        mesh=_sc_mesh(),
        compiler_params=pltpu.CompilerParams(use_tc_tiling_on_sc=False),
        scratch_types=[
            pltpu.VMEM((_GK, _CH), jnp.int32),
            pltpu.VMEM((_GK * _CH, _H), jnp.float32),
            pltpu.SemaphoreType.DMA,
        ],
    )


def _gather(tbl, idx):
    return _gather_kernel()(tbl, idx)


def _make_scatter(width):
    def body(msg_hbm, idx_hbm, zero_hbm, out_hbm, idx_v, row_v, acc_sh, sem):
        cid = lax.axis_index("c")
        sid = lax.axis_index("s")
        wid = sid * _NC + cid
        slab = pl.ds(sid * _NPS, _NPS)
        pltpu.sync_copy(zero_hbm.at[slab], acc_sh.at[slab])
        plsc.subcore_barrier()

        def fn(row0, k):
            pltpu.sync_copy(idx_hbm.at[pl.ds(row0, k)], idx_v.at[pl.ds(0, k)])
            pltpu.sync_copy(
                msg_hbm.at[pl.ds(pl.multiple_of(row0 * _CH, _CH), k * _CH)],
                row_v.at[pl.ds(0, k * _CH)],
            )
            descs = [
                pltpu.async_copy(
                    row_v.at[pl.ds(b * _CH, _CH)],
                    acc_sh.at[idx_v.at[b]],
                    sem,
                    add=True,
                )
                for b in range(k)
            ]
            for d in descs:
                d.wait()

        _sc_group_loop(wid, fn)

        plsc.subcore_barrier()
        pltpu.sync_copy(acc_sh.at[slab], out_hbm.at[cid].at[slab])

    return pl.kernel(
        body,
        out_type=jax.ShapeDtypeStruct((_NC, _N, width), jnp.float32),
        mesh=_sc_mesh(),
        compiler_params=pltpu.CompilerParams(use_tc_tiling_on_sc=False),
        scratch_types=[
            pltpu.VMEM((_GK, _CH), jnp.int32),
            pltpu.VMEM((_GK * _CH, width), jnp.float32),
            pltpu.VMEM_SHARED((_N, width), jnp.float32),
            pltpu.SemaphoreType.DMA,
        ],
    )


_make_scatter = functools.lru_cache(maxsize=None)(_make_scatter)


def _scatter32(msg, dst, zeros):
    return _make_scatter(32)(msg, dst, zeros)


def _scatter16(msg, dst, zeros):
    return _make_scatter(16)(msg, dst, zeros)


def _cnt_body(idx_hbm, ones_hbm, zero_hbm, out_hbm, idx_v, row_v, acc_sh, sem):
    cid = lax.axis_index("c")
    sid = lax.axis_index("s")
    wid = sid * _NC + cid
    slab = pl.ds(sid * _NPS, _NPS)
    pltpu.sync_copy(zero_hbm.at[slab], acc_sh.at[slab])
    pltpu.sync_copy(ones_hbm, row_v)
    plsc.subcore_barrier()

    def fn(row0, k):
        pltpu.sync_copy(idx_hbm.at[pl.ds(row0, k)], idx_v.at[pl.ds(0, k)])
        descs = [
            pltpu.async_copy(row_v, acc_sh.at[idx_v.at[b]], sem, add=True)
            for b in range(k)
        ]
        for d in descs:
            d.wait()

    _sc_group_loop(wid, fn)

    plsc.subcore_barrier()
    pltpu.sync_copy(acc_sh.at[slab], out_hbm.at[cid].at[slab])


@functools.lru_cache(maxsize=None)
def _cnt_kernel():
    return pl.kernel(
        _cnt_body,
        out_type=jax.ShapeDtypeStruct((_NC, _N, 16), jnp.float32),
        mesh=_sc_mesh(),
        compiler_params=pltpu.CompilerParams(use_tc_tiling_on_sc=False),
        scratch_types=[
            pltpu.VMEM((_GK, _CH), jnp.int32),
            pltpu.VMEM((_CH, 16), jnp.float32),
            pltpu.VMEM_SHARED((_N, 16), jnp.float32),
            pltpu.SemaphoreType.DMA,
        ],
    )


def _cnt(idx, ones_row, zeros):
    return _cnt_kernel()(idx, ones_row, zeros)


def _proj_body(x_ref, w_ref, b_ref, o_ref):
    o_ref[...] = (
        jnp.dot(x_ref[...], w_ref[...], preferred_element_type=jnp.float32)
        + b_ref[...]
    )


def _node_proj(x, w, b):
    return pl.pallas_call(
        _proj_body,
        out_shape=jax.ShapeDtypeStruct((_N, _H), jnp.float32),
        grid=(10,),
        in_specs=[
            pl.BlockSpec((_N // 10, _DF), lambda i: (i, 0)),
            pl.BlockSpec((_DF, _H), lambda i: (0, 0)),
            pl.BlockSpec((1, _H), lambda i: (0, 0)),
        ],
        out_specs=pl.BlockSpec((_N // 10, _H), lambda i: (i, 0)),
    )(x, w, b)


_MB = 640


def _make_msg_body(din, dout):
    def body(ea_ref, xj_ref, w1_ref, b1_ref, w2_ref, b2_ref, r_ref, o_ref):
        hid = jnp.maximum(
            jnp.dot(ea_ref[...], w1_ref[...], preferred_element_type=jnp.float32)
            + b1_ref[...],
            0.0,
        )
        we = (
            jnp.dot(hid, w2_ref[...], preferred_element_type=jnp.float32)
            + b2_ref[...]
        )
        x4 = jnp.dot(xj_ref[...], r_ref[...], preferred_element_type=jnp.float32)
        p = x4 * we
        w = din * dout
        while w > dout:
            w //= 2
            p = p[:, :w] + p[:, w:]
        o_ref[...] = p

    return body


def _msg(ea, xj, w1f, b1f, w2, b2, rmat, din, dout):
    return pl.pallas_call(
        _make_msg_body(din, dout),
        out_shape=jax.ShapeDtypeStruct((_E, dout), jnp.float32),
        grid=(_E // _MB,),
        in_specs=[
            pl.BlockSpec((_MB, _DE), lambda e: (e, 0)),
            pl.BlockSpec((_MB, din), lambda e: (e, 0)),
            pl.BlockSpec((_DE, 2 * _H), lambda e: (0, 0)),
            pl.BlockSpec((1, 2 * _H), lambda e: (0, 0)),
            pl.BlockSpec((2 * _H, din * dout), lambda e: (0, 0)),
            pl.BlockSpec((1, din * dout), lambda e: (0, 0)),
            pl.BlockSpec((din, din * dout), lambda e: (0, 0)),
        ],
        out_specs=pl.BlockSpec((_MB, dout), lambda e: (e, 0)),
    )(ea, xj, w1f, b1f, w2, b2, rmat)


def _make_upd_body(dout):
    def body(s2_ref, c2_ref, h_ref, root_ref, bias_ref, gam_ref, bet_ref, o_ref):
        s = s2_ref[0] + s2_ref[1]
        cnt = c2_ref[0][:, 0:1] + c2_ref[1][:, 0:1]
        hn = (
            s / jnp.maximum(cnt, 1.0)
            + jnp.dot(h_ref[...], root_ref[...], preferred_element_type=jnp.float32)
            + bias_ref[...]
        )
        mu = jnp.mean(hn, axis=0, keepdims=True)
        var = jnp.mean((hn - mu) * (hn - mu), axis=0, keepdims=True)
        bn = (hn - mu) * lax.rsqrt(var + 1e-5) * gam_ref[...] + bet_ref[...]
        o_ref[...] = jnp.maximum(bn, 0.0)

    return body


def _update(s2, c2, h, root, bias, gamma, beta, din, dout):
    return pl.pallas_call(
        _make_upd_body(dout),
        out_shape=jax.ShapeDtypeStruct((_N, dout), jnp.float32),
        in_specs=[
            pl.BlockSpec((_NC, _N, dout), lambda: (0, 0, 0)),
            pl.BlockSpec((_NC, _N, 16), lambda: (0, 0, 0)),
            pl.BlockSpec((_N, din), lambda: (0, 0)),
            pl.BlockSpec((din, dout), lambda: (0, 0)),
            pl.BlockSpec((1, dout), lambda: (0, 0)),
            pl.BlockSpec((1, dout), lambda: (0, 0)),
            pl.BlockSpec((1, dout), lambda: (0, 0)),
        ],
        out_specs=pl.BlockSpec((_N, dout), lambda: (0, 0)),
    )(s2, c2, h, root, bias, gamma, beta)


def _pool_body(h_ref, b_ref, fcw_ref, fcb_ref, o_ref):
    h = h_ref[...]
    onehot = (b_ref[...] == lax.broadcasted_iota(jnp.int32, (1, _NG), 1)).astype(
        jnp.float32
    )
    dn = (((0,), (0,)), ((), ()))
    ps = lax.dot_general(onehot, h, dn, preferred_element_type=jnp.float32)
    pc = lax.dot_general(
        onehot, jnp.ones((_N, 1), jnp.float32), dn, preferred_element_type=jnp.float32
    )
    pooled = ps / jnp.maximum(pc, 1.0)
    logits = (
        jnp.dot(pooled, fcw_ref[...], preferred_element_type=jnp.float32)
        + fcb_ref[...]
    )
    o_ref[...] = 1.0 / (1.0 + jnp.exp(-logits))


def _pool(h, batch2d, fcw, fcb):
    return pl.pallas_call(
        _pool_body,
        out_shape=jax.ShapeDtypeStruct((_NG, 12), jnp.float32),
        in_specs=[
            pl.BlockSpec((_N, 16), lambda: (0, 0)),
            pl.BlockSpec((_N, 1), lambda: (0, 0)),
            pl.BlockSpec((16, 12), lambda: (0, 0)),
            pl.BlockSpec((1, 12), lambda: (0, 0)),
        ],
        out_specs=pl.BlockSpec((_NG, 12), lambda: (0, 0)),
    )(h, batch2d, fcw, fcb)


def kernel(x, edge_index, edge_attr, batch, params):
    src = edge_index[0].reshape(_NCHUNKS, _CH)
    dst = edge_index[1].reshape(_NCHUNKS, _CH)

    h = _node_proj(x, params['node_W'], params['node_b'].reshape(1, _H))

    zeros16 = jnp.zeros((_N, 16), jnp.float32)
    ones_row = jnp.ones((_CH, 16), jnp.float32)
    c2 = _cnt(dst, ones_row, zeros16)

    zeros32 = jnp.zeros((_N, 32), jnp.float32)
    for i, (din, dout) in enumerate(_DIMS):
        lp = params['layers'][i]
        w1f = params['edge_W'] @ lp['W1']
        b1f = (params['edge_b'] @ lp['W1'] + lp['b1']).reshape(1, 2 * _H)
        xj = _gather(h, src)
        rmat = jnp.asarray(
            np.kron(np.eye(din, dtype=np.float32),
                    np.ones((1, dout), np.float32)))
        msg = _msg(edge_attr, xj, w1f, b1f, lp['W2'],
                   lp['b2'].reshape(1, din * dout), rmat, din, dout)
        scat = _scatter32 if dout == 32 else _scatter16
        s2 = scat(msg, dst, zeros32 if dout == 32 else zeros16)
        h = _update(s2, c2, h, lp['root'], lp['bias'].reshape(1, dout),
                    lp['gamma'].reshape(1, dout), lp['beta'].reshape(1, dout),
                    din, dout)

    return _pool(h, batch.reshape(_N, 1), params['fc_W'],
                 params['fc_b'].reshape(1, 12))
